# baseline (speedup 1.0000x reference)
import functools

import jax
import jax.numpy as jnp
from jax import lax
from jax.experimental import pallas as pl
from jax.experimental.pallas import tpu as pltpu
from jax.experimental.pallas import tpu_sc as plsc

_CROP = 7
_OFFSET = 0.5
_LEVEL_SHAPES = ((256, 256), (128, 128), (64, 64), (32, 32))
_PBASE = (0, 4, 8, 10)


def _prep(bboxes, table_rows):
    B, R = bboxes.shape[0], bboxes.shape[1]
    max_h = float(_LEVEL_SHAPES[0][0])
    max_w = float(_LEVEL_SHAPES[0][1])

    bw = bboxes[:, :, 3] - bboxes[:, :, 1]
    bh = bboxes[:, :, 2] - bboxes[:, :, 0]
    areas_sqrt = jnp.sqrt(bh * bw)
    levels = (jnp.floor(jnp.log(areas_sqrt / 224.0) / jnp.log(2.0)) + 4.0).astype(jnp.int32)
    levels = jnp.clip(levels, 1, 4)

    scale = jnp.power(2.0, levels.astype(jnp.float32))
    y0c = bboxes[:, :, 0] / scale
    x0c = bboxes[:, :, 1] / scale
    bh_s = bh / scale
    bw_s = bw / scale

    lvl_idx = levels - 1
    strides = jnp.power(2.0, lvl_idx.astype(jnp.float32))
    bdy = (max_h / strides - 1.0)[..., None]
    bdx = (max_w / strides - 1.0)[..., None]

    grid = (jnp.arange(_CROP, dtype=jnp.float32) + _OFFSET) / float(_CROP)
    gx = x0c[..., None] + grid[None, None, :] * bw_s[..., None]
    gy = y0c[..., None] + grid[None, None, :] * bh_s[..., None]
    x0 = jnp.minimum(jnp.maximum(0.0, jnp.floor(gx)), bdx)
    x1 = jnp.minimum(x0 + 1.0, bdx)
    y0 = jnp.minimum(jnp.maximum(0.0, jnp.floor(gy)), bdy)
    y1 = jnp.minimum(y0 + 1.0, bdy)
    lx = gx - x0
    hx = 1.0 - lx
    ly = gy - y0
    hy = 1.0 - ly

    sizes = [h * w for h, w in _LEVEL_SHAPES]
    offs = [0]
    for s in sizes[:-1]:
        offs.append(offs[-1] + s)
    batch_stride = offs[-1] + sizes[-1]
    widths = jnp.array([w for _, w in _LEVEL_SHAPES], jnp.int32)[lvl_idx]
    lvl_off = jnp.array(offs, jnp.int32)[lvl_idx]
    base = jnp.arange(B, dtype=jnp.int32)[:, None] * batch_stride + lvl_off

    xi0 = x0.astype(jnp.int32)
    xi1 = x1.astype(jnp.int32)
    yi0 = y0.astype(jnp.int32)
    yi1 = y1.astype(jnp.int32)

    N = B * R
    xx = jnp.stack([xi0, xi1], axis=-1).reshape(B, R, 14)
    yy = jnp.stack([yi0, yi1], axis=-1).reshape(B, R, 14)
    Fp = base[..., None] + yy * widths[..., None]

    x00 = xi0[..., 0]
    x16 = xi1[..., 6]
    ext = (x00 & 15) + (x16 - x00)
    fast = ext < 32
    med = jnp.logical_and(jnp.logical_not(fast), ext < 64)
    A0 = x00 >> 4

    psel = []
    for pb in _PBASE:
        psel.extend([pb, pb + 1, pb + 2, pb + 3])
    Fsel = Fp[..., jnp.array(psel, jnp.int32)].reshape(B, R, 4, 4)
    Ablk = (Fsel >> 4) + A0[..., None, None]
    maxblk = table_rows // 16 - 1
    fidx = jnp.stack([Ablk, jnp.minimum(Ablk + 1, maxblk)], axis=-1).reshape(N, 4, 8)
    tidx = (Fsel[..., None] + xx[..., None, None, :]).reshape(N, 4, 56)

    Am = (Fp.reshape(B, R, _CROP, 2) >> 4) + A0[..., None, None]
    midx = jnp.stack(
        [jnp.minimum(Am + u, maxblk) for u in range(4)], axis=-1
    ).reshape(N, _CROP, 8)

    pos = xx - (A0[..., None] << 4)
    pu = (pos >> 4).reshape(N, 14)
    pw = (pos & 15).reshape(N, 14)

    to_i32 = lambda a: lax.bitcast_convert_type(a, jnp.int32)
    rep16 = lambda a: jnp.repeat(a, 16, axis=-1)
    z16 = jnp.zeros((N, 16), jnp.int32)
    rows = [
        jnp.concatenate([fidx.reshape(N, 32), jnp.zeros((N, 96), jnp.int32)], 1),
        jnp.concatenate([tidx[:, 0], tidx[:, 1], z16], 1),
        jnp.concatenate([tidx[:, 2], tidx[:, 3], z16], 1),
        jnp.concatenate([rep16(to_i32(hy.reshape(N, _CROP))),
                         rep16((fast.astype(jnp.int32)
                                + 2 * med.astype(jnp.int32)).reshape(N)[:, None])], 1),
        jnp.concatenate([rep16(to_i32(ly.reshape(N, _CROP))), z16], 1),
        jnp.concatenate([rep16(to_i32(hx.reshape(N, _CROP))), z16], 1),
        jnp.concatenate([rep16(to_i32(lx.reshape(N, _CROP))), z16], 1),
        jnp.concatenate([rep16(pu[:, 0::2]), z16], 1),
        jnp.concatenate([rep16(pu[:, 1::2]), z16], 1),
        jnp.concatenate([rep16(pw[:, 0::2]), z16], 1),
        jnp.concatenate([rep16(pw[:, 1::2]), z16], 1),
        jnp.concatenate([midx.reshape(N, 56), jnp.zeros((N, 72), jnp.int32)], 1),
    ]
    meta = jnp.stack(rows, axis=1)

    info = plsc.get_sparse_core_info()
    NW = info.num_cores * info.num_subcores
    BPW = N // NW
    perm = (jnp.arange(BPW, dtype=jnp.int32)[None, :] * NW
            + jnp.arange(NW, dtype=jnp.int32)[:, None]).reshape(-1)
    return meta[perm]


def _roi_align_sc(table, meta):
    N = meta.shape[0]
    C = table.shape[1]
    nch = C // 16
    P = _CROP * _CROP

    info = plsc.get_sparse_core_info()
    NW = info.num_cores * info.num_subcores
    BPW = N // NW

    mesh = plsc.VectorSubcoreMesh(core_axis_name="c", subcore_axis_name="s")
    tb16 = table.reshape(-1, 16, C)

    @functools.partial(
        pl.kernel,
        mesh=mesh,
        compiler_params=pltpu.CompilerParams(
            use_tc_tiling_on_sc=False, needs_layout_passes=False),
        out_type=jax.ShapeDtypeStruct((N, P, C), jnp.float32),
        scratch_types=[
            pltpu.VMEM((BPW, 12, 128), jnp.int32),
            pltpu.VMEM((8, 16, C), jnp.float32),
            pltpu.VMEM((8, 16, C), jnp.float32),
            pltpu.VMEM((56, C), jnp.float32),
            pltpu.VMEM((P, C), jnp.float32),
            pltpu.SemaphoreType.DMA,
            pltpu.SemaphoreType.DMA,
        ],
    )
    def k(table_hbm, tb16_hbm, meta_hbm, out_hbm,
          meta_v, seg_a, seg_b, rows_v, out_v, gsem0, gsem1):
        wid = lax.axis_index("s") * info.num_cores + lax.axis_index("c")
        first = wid * BPW
        pltpu.sync_copy(meta_hbm.at[pl.ds(first, BPW)], meta_v)

        segs = (seg_a, seg_b)
        sems = (gsem0, gsem1)
        iota16 = lax.iota(jnp.int32, 16)

        def mvec(t, row, j):
            return meta_v[t, row, pl.ds(pl.multiple_of(j * 16, 16), 16)]

        def wvec(t, row, j):
            return plsc.bitcast(mvec(t, row, j), jnp.float32)

        def pair_rows(ip):
            return (1,) if ip == 3 else (0, 1)

        def out_row(ip, di):
            return 6 if ip == 3 else 2 * ip + di

        def combine(r_out, j, f00, f01, f10, f11, hyv, lyv, hxv, lxv, c):
            out_v[r_out * _CROP + j, pl.ds(c * 16, 16)] = (
                hyv * (hxv * f00 + lxv * f01) + lyv * (hxv * f10 + lxv * f11))

        def fast_compute(t, ip, b):
            seg = segs[b]
            for di in pair_rows(ip):
                r_out = out_row(ip, di)
                hyv = wvec(t, 3, r_out)
                lyv = wvec(t, 4, r_out)
                pl0 = 2 * di
                pl1 = 2 * di + 1

                def colf(j, cc):
                    hxv = wvec(t, 5, j)
                    lxv = wvec(t, 6, j)
                    pu0 = mvec(t, 7, j)
                    pu1 = mvec(t, 8, j)
                    pw0 = mvec(t, 9, j)
                    pw1 = mvec(t, 10, j)
                    i000 = pu0 + 2 * pl0
                    i001 = pu1 + 2 * pl0
                    i010 = pu0 + 2 * pl1
                    i011 = pu1 + 2 * pl1
                    for c in range(nch):
                        i3 = iota16 + c * 16
                        f00 = plsc.load_gather(seg, [i000, pw0, i3])
                        f01 = plsc.load_gather(seg, [i001, pw1, i3])
                        f10 = plsc.load_gather(seg, [i010, pw0, i3])
                        f11 = plsc.load_gather(seg, [i011, pw1, i3])
                        combine(r_out, j, f00, f01, f10, f11, hyv, lyv, hxv, lxv, c)
                    return cc

                lax.fori_loop(0, _CROP, colf, 0)

        def med_compute(t, i, b):
            seg = segs[b]
            hyv = wvec(t, 3, i)
            lyv = wvec(t, 4, i)

            def colm(j, cc):
                hxv = wvec(t, 5, j)
                lxv = wvec(t, 6, j)
                pu0 = mvec(t, 7, j)
                pu1 = mvec(t, 8, j)
                pw0 = mvec(t, 9, j)
                pw1 = mvec(t, 10, j)
                i010 = pu0 + 4
                i011 = pu1 + 4
                for c in range(nch):
                    i3 = iota16 + c * 16
                    f00 = plsc.load_gather(seg, [pu0, pw0, i3])
                    f01 = plsc.load_gather(seg, [pu1, pw1, i3])
                    f10 = plsc.load_gather(seg, [i010, pw0, i3])
                    f11 = plsc.load_gather(seg, [i011, pw1, i3])
                    combine(i, j, f00, f01, f10, f11, hyv, lyv, hxv, lxv, c)
                return cc

            lax.fori_loop(0, _CROP, colm, 0)

        def fb_compute(t, ip):
            for di in pair_rows(ip):
                r_out = out_row(ip, di)
                hyv = wvec(t, 3, r_out)
                lyv = wvec(t, 4, r_out)
                b0 = (2 * di) * 14
                b1 = (2 * di + 1) * 14

                def colb(j, cc):
                    hxv = wvec(t, 5, j)
                    lxv = wvec(t, 6, j)
                    for c in range(nch):
                        sl = pl.ds(c * 16, 16)
                        f00 = rows_v[b0 + 2 * j, sl]
                        f01 = rows_v[b0 + 2 * j + 1, sl]
                        f10 = rows_v[b1 + 2 * j, sl]
                        f11 = rows_v[b1 + 2 * j + 1, sl]
                        combine(r_out, j, f00, f01, f10, f11, hyv, lyv, hxv, lxv, c)
                    return cc

                lax.fori_loop(0, _CROP, colb, 0)

        def box(t, carry):
            flag = jnp.max(meta_v[t, 3, pl.ds(112, 16)])
            pos = t * NW + wid

            def fblocks(ip):
                return meta_v.at[t, 0, pl.ds(ip * 8, 8)]

            def mblocks(i):
                return meta_v.at[t, 11, pl.ds(i * 8, 8)]

            def ftaps(ip):
                return meta_v.at[t, 1 + ip // 2, pl.ds((ip % 2) * 56, 56)]

            @pl.when(flag == 1)
            def _fast():
                pltpu.make_async_copy(
                    tb16_hbm.at[fblocks(0)], seg_a, gsem0).start()
                for ip in range(4):
                    b = ip % 2
                    pltpu.make_async_copy(
                        tb16_hbm.at[fblocks(ip)], segs[b], sems[b]).wait()
                    if ip + 1 < 4:
                        nb = (ip + 1) % 2
                        pltpu.make_async_copy(
                            tb16_hbm.at[fblocks(ip + 1)], segs[nb], sems[nb]).start()
                    fast_compute(t, ip, b)

            @pl.when(flag == 2)
            def _med():
                pltpu.make_async_copy(
                    tb16_hbm.at[mblocks(0)], seg_a, gsem0).start()
                for i in range(_CROP):
                    b = i % 2
                    pltpu.make_async_copy(
                        tb16_hbm.at[mblocks(i)], segs[b], sems[b]).wait()
                    if i + 1 < _CROP:
                        nb = (i + 1) % 2
                        pltpu.make_async_copy(
                            tb16_hbm.at[mblocks(i + 1)], segs[nb], sems[nb]).start()
                    med_compute(t, i, b)

            @pl.when(flag == 0)
            def _fb():
                for ip in range(4):
                    cp = pltpu.make_async_copy(
                        table_hbm.at[ftaps(ip)], rows_v, gsem0)
                    cp.start()
                    cp.wait()
                    fb_compute(t, ip)

            pltpu.sync_copy(out_v, out_hbm.at[pos])
            return carry

        lax.fori_loop(0, BPW, box, 0)

    return k(table, tb16, meta)


def kernel(feat0, feat1, feat2, feat3, bboxes):
    B, _, _, C = feat0.shape
    R = bboxes.shape[1]
    table = jnp.concatenate(
        [f.reshape(B, -1, C) for f in (feat0, feat1, feat2, feat3)], axis=1
    ).reshape(-1, C)
    meta = _prep(bboxes, table.shape[0])
    out = _roi_align_sc(table, meta)
    return out.reshape(B, R, _CROP, _CROP, C)


# speedup vs baseline: 1.0412x; 1.0412x over previous
import functools

import jax
import jax.numpy as jnp
from jax import lax
from jax.experimental import pallas as pl
from jax.experimental.pallas import tpu as pltpu
from jax.experimental.pallas import tpu_sc as plsc

_CROP = 7
_OFFSET = 0.5
_LEVEL_SHAPES = ((256, 256), (128, 128), (64, 64), (32, 32))
_PBASE = (0, 4, 8, 10)


def _prep(bboxes, table_rows):
    B, R = bboxes.shape[0], bboxes.shape[1]
    max_h = float(_LEVEL_SHAPES[0][0])
    max_w = float(_LEVEL_SHAPES[0][1])

    bw = bboxes[:, :, 3] - bboxes[:, :, 1]
    bh = bboxes[:, :, 2] - bboxes[:, :, 0]
    areas_sqrt = jnp.sqrt(bh * bw)
    levels = (jnp.floor(jnp.log(areas_sqrt / 224.0) / jnp.log(2.0)) + 4.0).astype(jnp.int32)
    levels = jnp.clip(levels, 1, 4)

    scale = jnp.power(2.0, levels.astype(jnp.float32))
    y0c = bboxes[:, :, 0] / scale
    x0c = bboxes[:, :, 1] / scale
    bh_s = bh / scale
    bw_s = bw / scale

    lvl_idx = levels - 1
    strides = jnp.power(2.0, lvl_idx.astype(jnp.float32))
    bdy = (max_h / strides - 1.0)[..., None]
    bdx = (max_w / strides - 1.0)[..., None]

    grid = (jnp.arange(_CROP, dtype=jnp.float32) + _OFFSET) / float(_CROP)
    gx = x0c[..., None] + grid[None, None, :] * bw_s[..., None]
    gy = y0c[..., None] + grid[None, None, :] * bh_s[..., None]
    x0 = jnp.minimum(jnp.maximum(0.0, jnp.floor(gx)), bdx)
    x1 = jnp.minimum(x0 + 1.0, bdx)
    y0 = jnp.minimum(jnp.maximum(0.0, jnp.floor(gy)), bdy)
    y1 = jnp.minimum(y0 + 1.0, bdy)
    lx = gx - x0
    hx = 1.0 - lx
    ly = gy - y0
    hy = 1.0 - ly

    sizes = [h * w for h, w in _LEVEL_SHAPES]
    offs = [0]
    for s in sizes[:-1]:
        offs.append(offs[-1] + s)
    batch_stride = offs[-1] + sizes[-1]
    widths = jnp.array([w for _, w in _LEVEL_SHAPES], jnp.int32)[lvl_idx]
    lvl_off = jnp.array(offs, jnp.int32)[lvl_idx]
    base = jnp.arange(B, dtype=jnp.int32)[:, None] * batch_stride + lvl_off

    xi0 = x0.astype(jnp.int32)
    xi1 = x1.astype(jnp.int32)
    yi0 = y0.astype(jnp.int32)
    yi1 = y1.astype(jnp.int32)

    N = B * R
    xx = jnp.stack([xi0, xi1], axis=-1).reshape(B, R, 14)
    yy = jnp.stack([yi0, yi1], axis=-1).reshape(B, R, 14)
    Fp = base[..., None] + yy * widths[..., None]

    x00 = xi0[..., 0]
    x16 = xi1[..., 6]
    fast = ((x00 & 15) + (x16 - x00)) < 32
    A0 = x00 >> 4

    psel = []
    for pb in _PBASE:
        psel.extend([pb, pb + 1, pb + 2, pb + 3])
    Fsel = Fp[..., jnp.array(psel, jnp.int32)].reshape(B, R, 4, 4)
    Ablk = (Fsel >> 4) + A0[..., None, None]
    maxblk = table_rows // 16 - 1
    fidx = jnp.stack([Ablk, jnp.minimum(Ablk + 1, maxblk)], axis=-1).reshape(N, 4, 8)
    tidx = (Fsel[..., None] + xx[..., None, None, :]).reshape(N, 4, 56)

    pos = xx - (A0[..., None] << 4)
    pu = (pos >> 4).reshape(N, 14)
    pw = (pos & 15).reshape(N, 14)

    to_i32 = lambda a: lax.bitcast_convert_type(a, jnp.int32)
    rep16 = lambda a: jnp.repeat(a, 16, axis=-1)
    z16 = jnp.zeros((N, 16), jnp.int32)
    rows = [
        jnp.concatenate([fidx.reshape(N, 32), jnp.zeros((N, 96), jnp.int32)], 1),
        jnp.concatenate([tidx[:, 0], tidx[:, 1], z16], 1),
        jnp.concatenate([tidx[:, 2], tidx[:, 3], z16], 1),
        jnp.concatenate([rep16(to_i32(hy.reshape(N, _CROP))),
                         rep16(fast.astype(jnp.int32).reshape(N)[:, None])], 1),
        jnp.concatenate([rep16(to_i32(ly.reshape(N, _CROP))), z16], 1),
        jnp.concatenate([rep16(to_i32(hx.reshape(N, _CROP))), z16], 1),
        jnp.concatenate([rep16(to_i32(lx.reshape(N, _CROP))), z16], 1),
        jnp.concatenate([rep16(pu[:, 0::2]), z16], 1),
        jnp.concatenate([rep16(pu[:, 1::2]), z16], 1),
        jnp.concatenate([rep16(pw[:, 0::2]), z16], 1),
        jnp.concatenate([rep16(pw[:, 1::2]), z16], 1),
    ]
    meta = jnp.stack(rows, axis=1)

    info = plsc.get_sparse_core_info()
    NW = info.num_cores * info.num_subcores
    BPW = N // NW
    perm = (jnp.arange(BPW, dtype=jnp.int32)[None, :] * NW
            + jnp.arange(NW, dtype=jnp.int32)[:, None]).reshape(-1)
    return meta[perm]


def _roi_align_sc(table, meta):
    N = meta.shape[0]
    C = table.shape[1]
    nch = C // 16
    P = _CROP * _CROP

    info = plsc.get_sparse_core_info()
    NW = info.num_cores * info.num_subcores
    BPW = N // NW

    mesh = plsc.VectorSubcoreMesh(core_axis_name="c", subcore_axis_name="s")
    tb16 = table.reshape(-1, 16, C)

    @functools.partial(
        pl.kernel,
        mesh=mesh,
        compiler_params=pltpu.CompilerParams(
            use_tc_tiling_on_sc=False, needs_layout_passes=False),
        out_type=jax.ShapeDtypeStruct((N, P, C), jnp.float32),
        scratch_types=[
            pltpu.VMEM((BPW, 11, 128), jnp.int32),
            pltpu.VMEM((8, 16, C), jnp.float32),
            pltpu.VMEM((8, 16, C), jnp.float32),
            pltpu.VMEM((56, C), jnp.float32),
            pltpu.VMEM((2, P, C), jnp.float32),
            pltpu.SemaphoreType.DMA,
            pltpu.SemaphoreType.DMA,
            pltpu.SemaphoreType.DMA,
        ],
    )
    def k(table_hbm, tb16_hbm, meta_hbm, out_hbm,
          meta_v, seg_a, seg_b, rows_v, out_v, gsem0, gsem1, osem):
        wid = lax.axis_index("s") * info.num_cores + lax.axis_index("c")
        first = wid * BPW
        pltpu.sync_copy(meta_hbm.at[pl.ds(first, BPW)], meta_v)

        segs = (seg_a, seg_b)
        sems = (gsem0, gsem1)
        iota16 = lax.iota(jnp.int32, 16)

        def mvec(t, row, j):
            return meta_v[t, row, pl.ds(pl.multiple_of(j * 16, 16), 16)]

        def wvec(t, row, j):
            return plsc.bitcast(mvec(t, row, j), jnp.float32)

        def pair_rows(ip):
            return (1,) if ip == 3 else (0, 1)

        def out_row(ip, di):
            return 6 if ip == 3 else 2 * ip + di

        def combine(b2, r_out, j, f00, f01, f10, f11, hyv, lyv, hxv, lxv, c):
            out_v[b2, r_out * _CROP + j, pl.ds(c * 16, 16)] = (
                hyv * (hxv * f00 + lxv * f01) + lyv * (hxv * f10 + lxv * f11))

        def fast_compute(t, b2, ip, b):
            seg = segs[b]
            for di in pair_rows(ip):
                r_out = out_row(ip, di)
                hyv = wvec(t, 3, r_out)
                lyv = wvec(t, 4, r_out)
                pl0 = 2 * di
                pl1 = 2 * di + 1

                def colf(j, cc):
                    hxv = wvec(t, 5, j)
                    lxv = wvec(t, 6, j)
                    pu0 = mvec(t, 7, j)
                    pu1 = mvec(t, 8, j)
                    pw0 = mvec(t, 9, j)
                    pw1 = mvec(t, 10, j)
                    i000 = pu0 + 2 * pl0
                    i001 = pu1 + 2 * pl0
                    i010 = pu0 + 2 * pl1
                    i011 = pu1 + 2 * pl1
                    for c in range(nch):
                        i3 = iota16 + c * 16
                        f00 = plsc.load_gather(seg, [i000, pw0, i3])
                        f01 = plsc.load_gather(seg, [i001, pw1, i3])
                        f10 = plsc.load_gather(seg, [i010, pw0, i3])
                        f11 = plsc.load_gather(seg, [i011, pw1, i3])
                        combine(b2, r_out, j, f00, f01, f10, f11, hyv, lyv, hxv, lxv, c)
                    return cc

                lax.fori_loop(0, _CROP, colf, 0)

        def fb_compute(t, b2, ip):
            for di in pair_rows(ip):
                r_out = out_row(ip, di)
                hyv = wvec(t, 3, r_out)
                lyv = wvec(t, 4, r_out)
                b0 = (2 * di) * 14
                b1 = (2 * di + 1) * 14

                def colb(j, cc):
                    hxv = wvec(t, 5, j)
                    lxv = wvec(t, 6, j)
                    for c in range(nch):
                        sl = pl.ds(c * 16, 16)
                        f00 = rows_v[b0 + 2 * j, sl]
                        f01 = rows_v[b0 + 2 * j + 1, sl]
                        f10 = rows_v[b1 + 2 * j, sl]
                        f11 = rows_v[b1 + 2 * j + 1, sl]
                        combine(b2, r_out, j, f00, f01, f10, f11, hyv, lyv, hxv, lxv, c)
                    return cc

                lax.fori_loop(0, _CROP, colb, 0)

        def box(t, carry):
            flag = jnp.max(meta_v[t, 3, pl.ds(112, 16)])
            b2 = lax.rem(t, 2)
            pos = t * NW + wid

            @pl.when(t >= 2)
            def _drain():
                pltpu.make_async_copy(
                    out_v.at[b2], out_hbm.at[pos - 2 * NW], osem).wait()

            def fblocks(ip):
                return meta_v.at[t, 0, pl.ds(ip * 8, 8)]

            def ftaps(ip):
                return meta_v.at[t, 1 + ip // 2, pl.ds((ip % 2) * 56, 56)]

            @pl.when(flag == 1)
            def _fast():
                pltpu.make_async_copy(
                    tb16_hbm.at[fblocks(0)], seg_a, gsem0).start()
                for ip in range(4):
                    b = ip % 2
                    pltpu.make_async_copy(
                        tb16_hbm.at[fblocks(ip)], segs[b], sems[b]).wait()
                    if ip + 1 < 4:
                        nb = (ip + 1) % 2
                        pltpu.make_async_copy(
                            tb16_hbm.at[fblocks(ip + 1)], segs[nb], sems[nb]).start()
                    fast_compute(t, b2, ip, b)

            @pl.when(flag == 0)
            def _fb():
                for ip in range(4):
                    cp = pltpu.make_async_copy(
                        table_hbm.at[ftaps(ip)], rows_v, gsem0)
                    cp.start()
                    cp.wait()
                    fb_compute(t, b2, ip)

            pltpu.make_async_copy(out_v.at[b2], out_hbm.at[pos], osem).start()
            return carry

        lax.fori_loop(0, BPW, box, 0)
        pltpu.make_async_copy(
            out_v.at[0], out_hbm.at[(BPW - 2) * NW + wid], osem).wait()
        pltpu.make_async_copy(
            out_v.at[1], out_hbm.at[(BPW - 1) * NW + wid], osem).wait()

    return k(table, tb16, meta)


def kernel(feat0, feat1, feat2, feat3, bboxes):
    B, _, _, C = feat0.shape
    R = bboxes.shape[1]
    table = jnp.concatenate(
        [f.reshape(B, -1, C) for f in (feat0, feat1, feat2, feat3)], axis=1
    ).reshape(-1, C)
    meta = _prep(bboxes, table.shape[0])
    out = _roi_align_sc(table, meta)
    return out.reshape(B, R, _CROP, _CROP, C)
